# initial kernel scaffold (unmeasured)
import jax
import jax.numpy as jnp
from jax import lax
from jax.experimental import pallas as pl
from jax.experimental.pallas import tpu as pltpu


def kernel(
    x,
):
    def body(*refs):
        pass

    out_shape = jax.ShapeDtypeStruct(..., jnp.float32)
    return pl.pallas_call(body, out_shape=out_shape)(...)


# baseline (device time: 79805 ns/iter reference)
import jax
import jax.numpy as jnp
from jax import lax
from jax.experimental import pallas as pl
from jax.experimental.pallas import tpu as pltpu

N_Z = 4


def kernel(x):
    m, n_full = x.shape
    n = n_full // N_Z

    def body(x_ref, out_ref, send_sems, recv_sems):
        my_x = lax.axis_index("x")
        my_y = lax.axis_index("y")
        my_z = lax.axis_index("z")

        barrier_sem = pltpu.get_barrier_semaphore()
        for d in range(1, N_Z):
            q = lax.rem(my_z + d, N_Z)
            pl.semaphore_signal(
                barrier_sem, inc=1,
                device_id=(my_x, my_y, q),
                device_id_type=pl.DeviceIdType.MESH,
            )
        pl.semaphore_wait(barrier_sem, N_Z - 1)

        out_ref[pl.ds(my_z * m, m), :] = x_ref[:, pl.ds(my_z * n, n)]

        rdmas = []
        for d in range(1, N_Z):
            q = lax.rem(my_z + d, N_Z)
            rdma = pltpu.make_async_remote_copy(
                src_ref=x_ref.at[:, pl.ds(q * n, n)],
                dst_ref=out_ref.at[pl.ds(my_z * m, m), :],
                send_sem=send_sems.at[d - 1],
                recv_sem=recv_sems.at[(N_Z - 1) - d],
                device_id=(my_x, my_y, q),
                device_id_type=pl.DeviceIdType.MESH,
            )
            rdma.start()
            rdmas.append(rdma)

        for rdma in rdmas:
            rdma.wait()

    return pl.pallas_call(
        body,
        out_shape=jax.ShapeDtypeStruct((N_Z * m, n), x.dtype),
        in_specs=[pl.BlockSpec(memory_space=pltpu.VMEM)],
        out_specs=pl.BlockSpec(memory_space=pltpu.VMEM),
        scratch_shapes=[
            pltpu.SemaphoreType.DMA((N_Z - 1,)),
            pltpu.SemaphoreType.DMA((N_Z - 1,)),
        ],
        compiler_params=pltpu.CompilerParams(collective_id=0),
    )(x)


# device time: 70942 ns/iter; 1.1249x vs baseline; 1.1249x over previous
import jax
import jax.numpy as jnp
from jax import lax
from jax.experimental import pallas as pl
from jax.experimental.pallas import tpu as pltpu

N_Z = 4


def kernel(x):
    m, n_full = x.shape
    n = n_full // N_Z
    hm = m // 2

    def body(x_ref, out_ref, z_send_sems, z_recv_sems, x_send_sems, x_recv_sems):
        my_x = lax.axis_index("x")
        my_y = lax.axis_index("y")
        my_z = lax.axis_index("z")

        barrier_sem = pltpu.get_barrier_semaphore()
        for d in range(1, N_Z):
            q = lax.rem(my_z + d, N_Z)
            pl.semaphore_signal(
                barrier_sem, inc=1,
                device_id=(my_x, my_y, q),
                device_id_type=pl.DeviceIdType.MESH,
            )
        pl.semaphore_signal(
            barrier_sem, inc=1,
            device_id=(1 - my_x, my_y, my_z),
            device_id_type=pl.DeviceIdType.MESH,
        )
        pl.semaphore_wait(barrier_sem, N_Z)

        z_rdmas = {}
        for d in range(1, N_Z):
            q = lax.rem(my_z + d, N_Z)
            rdma = pltpu.make_async_remote_copy(
                src_ref=x_ref.at[pl.ds(my_x * hm, hm), pl.ds(q * n, n)],
                dst_ref=out_ref.at[pl.ds(my_z * m + my_x * hm, hm), :],
                send_sem=z_send_sems.at[d - 1],
                recv_sem=z_recv_sems.at[(N_Z - 1) - d],
                device_id=(my_x, my_y, q),
                device_id_type=pl.DeviceIdType.MESH,
            )
            rdma.start()
            z_rdmas[d] = rdma

        out_ref[pl.ds(my_z * m, m), :] = x_ref[:, pl.ds(my_z * n, n)]

        fwds = []
        for s in range(N_Z - 1):
            z_rdmas[(N_Z - 1) - s].wait_recv()
            p = lax.rem(my_z + s + 1, N_Z)
            fwd = pltpu.make_async_remote_copy(
                src_ref=out_ref.at[pl.ds(p * m + my_x * hm, hm), :],
                dst_ref=out_ref.at[pl.ds(p * m + my_x * hm, hm), :],
                send_sem=x_send_sems.at[s],
                recv_sem=x_recv_sems.at[s],
                device_id=(1 - my_x, my_y, my_z),
                device_id_type=pl.DeviceIdType.MESH,
            )
            fwd.start()
            fwds.append(fwd)

        for fwd in fwds:
            fwd.wait_recv()

        for d in range(1, N_Z):
            z_rdmas[d].wait_send()
        for fwd in fwds:
            fwd.wait_send()

    return pl.pallas_call(
        body,
        out_shape=jax.ShapeDtypeStruct((N_Z * m, n), x.dtype),
        in_specs=[pl.BlockSpec(memory_space=pltpu.VMEM)],
        out_specs=pl.BlockSpec(memory_space=pltpu.VMEM),
        scratch_shapes=[
            pltpu.SemaphoreType.DMA((N_Z - 1,)),
            pltpu.SemaphoreType.DMA((N_Z - 1,)),
            pltpu.SemaphoreType.DMA((N_Z - 1,)),
            pltpu.SemaphoreType.DMA((N_Z - 1,)),
        ],
        compiler_params=pltpu.CompilerParams(collective_id=0),
    )(x)
